# baseline (speedup 1.0000x reference)
import jax
import jax.numpy as jnp
from jax import lax
from jax.experimental import pallas as pl
from jax.experimental.pallas import tpu as pltpu
from jax.experimental.pallas import tpu_sc as plsc

_N_CAT = 26
_VOCAB = 100000
_DIM = 32
_BATCH = 16384

_NUM_CORES = 2
_NUM_SUBCORES = 16
_NUM_WORKERS = _NUM_CORES * _NUM_SUBCORES
_CHUNK = _BATCH // _NUM_WORKERS
_GSUB = 128
_NG = _CHUNK // _GSUB


def _sc_body(*refs):
    idx_refs = refs[:_N_CAT]
    tab_refs = refs[_N_CAT:2 * _N_CAT]
    out_ref = refs[2 * _N_CAT]
    idx_v, rows_v, gsems, wsems = refs[2 * _N_CAT + 1:]

    wid = lax.axis_index("s") * _NUM_CORES + lax.axis_index("c")
    base = wid * _CHUNK
    row0 = wid * _NG

    gather_handles = [None, None]
    write_handles = [None, None]

    def start_table(i):
        b = i % 2
        pltpu.sync_copy(idx_refs[i].at[pl.ds(row0, _NG)], idx_v.at[b])
        hs = []
        for j in range(_NG):
            hs.append(
                pltpu.async_copy(
                    tab_refs[i].at[idx_v.at[b, j]],
                    rows_v.at[b, pl.ds(j * _GSUB, _GSUB)],
                    gsems.at[b],
                )
            )
        gather_handles[b] = hs

    def finish_table(i):
        b = i % 2
        for h in gather_handles[b]:
            h.wait()
        write_handles[b] = pltpu.async_copy(
            rows_v.at[b], out_ref.at[pl.ds(base, _CHUNK), i], wsems.at[b]
        )

    start_table(0)
    for i in range(1, _N_CAT):
        b = i % 2
        if write_handles[b] is not None:
            write_handles[b].wait()
        start_table(i)
        finish_table(i - 1)
    finish_table(_N_CAT - 1)
    write_handles[0].wait()
    write_handles[1].wait()


@jax.jit
def _tabular_embedding(idx_list, tab_list):
    mesh = plsc.VectorSubcoreMesh(
        core_axis_name="c", subcore_axis_name="s",
        num_cores=_NUM_CORES, num_subcores=_NUM_SUBCORES,
    )
    out = pl.kernel(
        _sc_body,
        out_type=jax.ShapeDtypeStruct((_BATCH, _N_CAT, _DIM), jnp.float32),
        mesh=mesh,
        compiler_params=pltpu.CompilerParams(use_tc_tiling_on_sc=False),
        scratch_types=[
            pltpu.VMEM((2, _NG, _GSUB), jnp.int32),
            pltpu.VMEM((2, _CHUNK, _DIM), jnp.float32),
            pltpu.SemaphoreType.DMA((2,)),
            pltpu.SemaphoreType.DMA((2,)),
        ],
    )(*idx_list, *tab_list)
    return out.reshape(_BATCH, _N_CAT * _DIM)


def kernel(cat_0, cat_1, cat_2, cat_3, cat_4, cat_5, cat_6, cat_7, cat_8,
           cat_9, cat_10, cat_11, cat_12, cat_13, cat_14, cat_15, cat_16,
           cat_17, cat_18, cat_19, cat_20, cat_21, cat_22, cat_23, cat_24,
           cat_25, W_0, W_1, W_2, W_3, W_4, W_5, W_6, W_7, W_8, W_9, W_10,
           W_11, W_12, W_13, W_14, W_15, W_16, W_17, W_18, W_19, W_20,
           W_21, W_22, W_23, W_24, W_25):
    args = dict(locals())
    idx_list = [
        args[f"cat_{i}"].astype(jnp.int32).reshape(_BATCH // _GSUB, _GSUB)
        for i in range(_N_CAT)
    ]
    tab_list = [args[f"W_{i}"] for i in range(_N_CAT)]
    return _tabular_embedding(idx_list, tab_list)


# speedup vs baseline: 3.3123x; 3.3123x over previous
import jax
import jax.numpy as jnp
from jax import lax
from jax.experimental import pallas as pl
from jax.experimental.pallas import tpu as pltpu
from jax.experimental.pallas import tpu_sc as plsc

_N_CAT = 26
_VOCAB = 100000
_DIM = 32
_BATCH = 16384

_NUM_CORES = 2
_NUM_SUBCORES = 16
_NUM_WORKERS = _NUM_CORES * _NUM_SUBCORES
_LANES = 16
_OCHUNK = 8192
_NCHUNK = _BATCH // _OCHUNK
_STEPS = _OCHUNK // _LANES


def _sc_body(*refs):
    idx_refs = refs[:_N_CAT]
    tab_refs = refs[_N_CAT:2 * _N_CAT]
    out_ref = refs[2 * _N_CAT]
    src_v, idx_v, out_v = refs[2 * _N_CAT + 1:]

    w = lax.axis_index("s") * _NUM_CORES + lax.axis_index("c")
    g_src = w // 8
    s_src = w % 8

    for i in range(_N_CAT):
        pltpu.sync_copy(idx_refs[i], idx_v)
        pltpu.sync_copy(tab_refs[i].at[g_src, s_src], src_v)

        def chunk(h):
            def step(j, _):
                iv = idx_v[pl.ds(h * _OCHUNK + j * _LANES, _LANES)]
                out_v[pl.ds(j * _LANES, _LANES)] = plsc.load_gather(
                    src_v, [iv])
                return _

            lax.fori_loop(0, _STEPS, step, None)
            pltpu.sync_copy(
                out_v,
                out_ref.at[i * 4 + g_src, s_src, pl.ds(h * _OCHUNK, _OCHUNK)],
            )

        for h in range(_NCHUNK):
            chunk(h)


@jax.jit
def _tabular_embedding(idx_list, tab_list):
    mesh = plsc.VectorSubcoreMesh(
        core_axis_name="c", subcore_axis_name="s",
        num_cores=_NUM_CORES, num_subcores=_NUM_SUBCORES,
    )
    out_t = pl.kernel(
        _sc_body,
        out_type=jax.ShapeDtypeStruct((_N_CAT * 4, 8, _BATCH), jnp.float32),
        mesh=mesh,
        compiler_params=pltpu.CompilerParams(
            use_tc_tiling_on_sc=True, needs_layout_passes=False),
        scratch_types=[
            pltpu.VMEM((_VOCAB,), jnp.float32),
            pltpu.VMEM((_BATCH,), jnp.int32),
            pltpu.VMEM((_OCHUNK,), jnp.float32),
        ],
    )(*idx_list, *tab_list)
    return out_t.reshape(_N_CAT * _DIM, _BATCH).T


def kernel(cat_0, cat_1, cat_2, cat_3, cat_4, cat_5, cat_6, cat_7, cat_8,
           cat_9, cat_10, cat_11, cat_12, cat_13, cat_14, cat_15, cat_16,
           cat_17, cat_18, cat_19, cat_20, cat_21, cat_22, cat_23, cat_24,
           cat_25, W_0, W_1, W_2, W_3, W_4, W_5, W_6, W_7, W_8, W_9, W_10,
           W_11, W_12, W_13, W_14, W_15, W_16, W_17, W_18, W_19, W_20,
           W_21, W_22, W_23, W_24, W_25):
    args = dict(locals())
    idx_list = [args[f"cat_{i}"].astype(jnp.int32) for i in range(_N_CAT)]
    tab_list = [
        args[f"W_{i}"].T.reshape(4, 8, _VOCAB) for i in range(_N_CAT)
    ]
    return _tabular_embedding(idx_list, tab_list)


# speedup vs baseline: 5.4406x; 1.6426x over previous
import jax
import jax.numpy as jnp
from jax import lax
from jax.experimental import pallas as pl
from jax.experimental.pallas import tpu as pltpu
from jax.experimental.pallas import tpu_sc as plsc

_N_CAT = 26
_VOCAB = 100000
_DIM = 32
_BATCH = 16384

_NUM_CORES = 2
_NUM_SUBCORES = 16
_NUM_WORKERS = _NUM_CORES * _NUM_SUBCORES
_LANES = 16
_OCHUNK = 8192
_NCHUNK = _BATCH // _OCHUNK
_STEPS = _OCHUNK // _LANES


def _sc_body(*refs):
    idx_refs = refs[:_N_CAT]
    tab_refs = refs[_N_CAT:2 * _N_CAT]
    out_ref = refs[2 * _N_CAT]
    src_v, idx_v, out_v = refs[2 * _N_CAT + 1:]

    w = lax.axis_index("s") * _NUM_CORES + lax.axis_index("c")
    g_src = w // 8
    s_src = w % 8

    for i in range(_N_CAT):
        pltpu.sync_copy(idx_refs[i], idx_v)
        pltpu.sync_copy(tab_refs[i].at[g_src, s_src], src_v)

        def chunk(h):
            @plsc.parallel_loop(0, _OCHUNK, step=_LANES, unroll=8)
            def _gather(j):
                iv = idx_v[pl.ds(h * _OCHUNK + j, _LANES)]
                out_v[pl.ds(j, _LANES)] = plsc.load_gather(src_v, [iv])

            pltpu.sync_copy(
                out_v,
                out_ref.at[i * 4 + g_src, s_src, pl.ds(h * _OCHUNK, _OCHUNK)],
            )

        for h in range(_NCHUNK):
            chunk(h)


@jax.jit
def _tabular_embedding(idx_list, tab_list):
    mesh = plsc.VectorSubcoreMesh(
        core_axis_name="c", subcore_axis_name="s",
        num_cores=_NUM_CORES, num_subcores=_NUM_SUBCORES,
    )
    out_t = pl.kernel(
        _sc_body,
        out_type=jax.ShapeDtypeStruct((_N_CAT * 4, 8, _BATCH), jnp.float32),
        mesh=mesh,
        compiler_params=pltpu.CompilerParams(
            use_tc_tiling_on_sc=True, needs_layout_passes=False),
        scratch_types=[
            pltpu.VMEM((_VOCAB,), jnp.float32),
            pltpu.VMEM((_BATCH,), jnp.int32),
            pltpu.VMEM((_OCHUNK,), jnp.float32),
        ],
    )(*idx_list, *tab_list)
    return out_t.reshape(_N_CAT * _DIM, _BATCH).T


def kernel(cat_0, cat_1, cat_2, cat_3, cat_4, cat_5, cat_6, cat_7, cat_8,
           cat_9, cat_10, cat_11, cat_12, cat_13, cat_14, cat_15, cat_16,
           cat_17, cat_18, cat_19, cat_20, cat_21, cat_22, cat_23, cat_24,
           cat_25, W_0, W_1, W_2, W_3, W_4, W_5, W_6, W_7, W_8, W_9, W_10,
           W_11, W_12, W_13, W_14, W_15, W_16, W_17, W_18, W_19, W_20,
           W_21, W_22, W_23, W_24, W_25):
    args = dict(locals())
    idx_list = [args[f"cat_{i}"].astype(jnp.int32) for i in range(_N_CAT)]
    tab_list = [
        args[f"W_{i}"].T.reshape(4, 8, _VOCAB) for i in range(_N_CAT)
    ]
    return _tabular_embedding(idx_list, tab_list)


# speedup vs baseline: 5.4919x; 1.0094x over previous
import jax
import jax.numpy as jnp
from jax import lax
from jax.experimental import pallas as pl
from jax.experimental.pallas import tpu as pltpu
from jax.experimental.pallas import tpu_sc as plsc

_N_CAT = 26
_VOCAB = 100000
_DIM = 32
_BATCH = 16384

_NUM_CORES = 2
_NUM_SUBCORES = 16
_NUM_WORKERS = _NUM_CORES * _NUM_SUBCORES
_LANES = 16
_OCHUNK = 4096
_NCHUNK = _BATCH // _OCHUNK
_NRC = 1


def _sc_body(*refs):
    idx_refs = refs[:_N_CAT]
    tab_refs = refs[_N_CAT:2 * _N_CAT]
    out_ref = refs[2 * _N_CAT]
    src_v, idx_v, out_v, isem, rsems, osems = refs[2 * _N_CAT + 1:]

    w = lax.axis_index("s") * _NUM_CORES + lax.axis_index("c")
    g_src = w // 8
    s_src = w % 8

    def start_loads(i):
        ih = pltpu.async_copy(idx_refs[i], idx_v, isem)
        rhs = [
            pltpu.async_copy(
                tab_refs[i].at[g_src, s_src], src_v, rsems.at[k])
            for k in range(_NRC)
        ]
        return ih, rhs

    loads = start_loads(0)
    write_handles = [None, None]
    for i in range(_N_CAT):
        ih, rhs = loads
        ih.wait()
        for h_ in rhs:
            h_.wait()
        for h in range(_NCHUNK):
            ob = h % 2
            if write_handles[ob] is not None:
                write_handles[ob].wait()

            @plsc.parallel_loop(0, _OCHUNK, step=_LANES, unroll=8)
            def _gather(j):
                iv = idx_v[pl.ds(h * _OCHUNK + j, _LANES)]
                out_v[ob, pl.ds(j, _LANES)] = plsc.load_gather(src_v, [iv])

            if h == _NCHUNK - 1 and i + 1 < _N_CAT:
                loads = start_loads(i + 1)
            write_handles[ob] = pltpu.async_copy(
                out_v.at[ob],
                out_ref.at[i * 4 + g_src, s_src, pl.ds(h * _OCHUNK, _OCHUNK)],
                osems.at[ob],
            )
    write_handles[0].wait()
    write_handles[1].wait()


@jax.jit
def _tabular_embedding(idx_list, tab_list):
    mesh = plsc.VectorSubcoreMesh(
        core_axis_name="c", subcore_axis_name="s",
        num_cores=_NUM_CORES, num_subcores=_NUM_SUBCORES,
    )
    out_t = pl.kernel(
        _sc_body,
        out_type=jax.ShapeDtypeStruct((_N_CAT * 4, 8, _BATCH), jnp.float32),
        mesh=mesh,
        compiler_params=pltpu.CompilerParams(
            use_tc_tiling_on_sc=True, needs_layout_passes=False),
        scratch_types=[
            pltpu.VMEM((_VOCAB,), jnp.float32),
            pltpu.VMEM((_BATCH,), jnp.int32),
            pltpu.VMEM((2, _OCHUNK), jnp.float32),
            pltpu.SemaphoreType.DMA,
            pltpu.SemaphoreType.DMA((_NRC,)),
            pltpu.SemaphoreType.DMA((2,)),
        ],
    )(*idx_list, *tab_list)
    return out_t.reshape(_N_CAT * _DIM, _BATCH).T


def kernel(cat_0, cat_1, cat_2, cat_3, cat_4, cat_5, cat_6, cat_7, cat_8,
           cat_9, cat_10, cat_11, cat_12, cat_13, cat_14, cat_15, cat_16,
           cat_17, cat_18, cat_19, cat_20, cat_21, cat_22, cat_23, cat_24,
           cat_25, W_0, W_1, W_2, W_3, W_4, W_5, W_6, W_7, W_8, W_9, W_10,
           W_11, W_12, W_13, W_14, W_15, W_16, W_17, W_18, W_19, W_20,
           W_21, W_22, W_23, W_24, W_25):
    args = dict(locals())
    idx_list = [args[f"cat_{i}"].astype(jnp.int32) for i in range(_N_CAT)]
    tab_list = [
        args[f"W_{i}"].T.reshape(4, 8, _VOCAB) for i in range(_N_CAT)
    ]
    return _tabular_embedding(idx_list, tab_list)


# speedup vs baseline: 6.3543x; 1.1570x over previous
import jax
import jax.numpy as jnp
from jax import lax
from jax.experimental import pallas as pl
from jax.experimental.pallas import tpu as pltpu
from jax.experimental.pallas import tpu_sc as plsc

_N_CAT = 26
_VOCAB = 100000
_DIM = 32
_BATCH = 16384

_NUM_CORES = 2
_NUM_SUBCORES = 16
_NUM_WORKERS = _NUM_CORES * _NUM_SUBCORES
_LANES = 16
_OCHUNK = 4096
_NCHUNK = _BATCH // _OCHUNK
_NRC = 1


def _sc_body(*refs):
    idx_refs = refs[:_N_CAT]
    tab_refs = refs[_N_CAT:2 * _N_CAT]
    out_ref = refs[2 * _N_CAT]
    src_v, idx_v, out_v, idx_sh, isem, rsems, osems, ssem = \
        refs[2 * _N_CAT + 1:]

    sid = lax.axis_index("s")
    w = sid * _NUM_CORES + lax.axis_index("c")
    g_src = w // 8
    s_src = w % 8

    def stager_of(i):
        return i % _NUM_SUBCORES

    for i in range(2):
        @pl.when(sid == stager_of(i))
        def _stage0():
            pltpu.sync_copy(idx_refs[i], idx_sh.at[i])
    plsc.subcore_barrier()

    def start_loads(i):
        ih = pltpu.async_copy(idx_sh.at[i % 2], idx_v, isem)
        rhs = [
            pltpu.async_copy(
                tab_refs[i].at[g_src, s_src], src_v, rsems.at[k])
            for k in range(_NRC)
        ]
        return ih, rhs

    loads = start_loads(0)
    write_handles = [None, None]
    for i in range(_N_CAT):
        ih, rhs = loads
        ih.wait()
        for h_ in rhs:
            h_.wait()
        if 1 <= i and i + 1 < _N_CAT:
            @pl.when(sid == stager_of(i + 1))
            def _stage_wait():
                pltpu.make_async_copy(
                    idx_refs[i + 1], idx_sh.at[(i + 1) % 2], ssem).wait()
        plsc.subcore_barrier()
        if i + 2 < _N_CAT:
            @pl.when(sid == stager_of(i + 2))
            def _stage_next():
                pltpu.async_copy(
                    idx_refs[i + 2], idx_sh.at[(i + 2) % 2], ssem)
        for h in range(_NCHUNK):
            ob = h % 2
            if write_handles[ob] is not None:
                write_handles[ob].wait()

            @plsc.parallel_loop(0, _OCHUNK, step=_LANES, unroll=8)
            def _gather(j):
                iv = idx_v[pl.ds(h * _OCHUNK + j, _LANES)]
                out_v[ob, pl.ds(j, _LANES)] = plsc.load_gather(src_v, [iv])

            if h == _NCHUNK - 1 and i + 1 < _N_CAT:
                loads = start_loads(i + 1)
            write_handles[ob] = pltpu.async_copy(
                out_v.at[ob],
                out_ref.at[i * 4 + g_src, s_src, pl.ds(h * _OCHUNK, _OCHUNK)],
                osems.at[ob],
            )
    write_handles[0].wait()
    write_handles[1].wait()


@jax.jit
def _tabular_embedding(idx_list, tab_list):
    mesh = plsc.VectorSubcoreMesh(
        core_axis_name="c", subcore_axis_name="s",
        num_cores=_NUM_CORES, num_subcores=_NUM_SUBCORES,
    )
    out_t = pl.kernel(
        _sc_body,
        out_type=jax.ShapeDtypeStruct((_N_CAT * 4, 8, _BATCH), jnp.float32),
        mesh=mesh,
        compiler_params=pltpu.CompilerParams(
            use_tc_tiling_on_sc=True, needs_layout_passes=False),
        scratch_types=[
            pltpu.VMEM((_VOCAB,), jnp.float32),
            pltpu.VMEM((_BATCH,), jnp.int32),
            pltpu.VMEM((2, _OCHUNK), jnp.float32),
            pltpu.VMEM_SHARED((2, _BATCH), jnp.int32),
            pltpu.SemaphoreType.DMA,
            pltpu.SemaphoreType.DMA((_NRC,)),
            pltpu.SemaphoreType.DMA((2,)),
            pltpu.SemaphoreType.DMA,
        ],
    )(*idx_list, *tab_list)
    return out_t.reshape(_N_CAT * _DIM, _BATCH).T


def kernel(cat_0, cat_1, cat_2, cat_3, cat_4, cat_5, cat_6, cat_7, cat_8,
           cat_9, cat_10, cat_11, cat_12, cat_13, cat_14, cat_15, cat_16,
           cat_17, cat_18, cat_19, cat_20, cat_21, cat_22, cat_23, cat_24,
           cat_25, W_0, W_1, W_2, W_3, W_4, W_5, W_6, W_7, W_8, W_9, W_10,
           W_11, W_12, W_13, W_14, W_15, W_16, W_17, W_18, W_19, W_20,
           W_21, W_22, W_23, W_24, W_25):
    args = dict(locals())
    idx_list = [args[f"cat_{i}"].astype(jnp.int32) for i in range(_N_CAT)]
    tab_list = [
        args[f"W_{i}"].T.reshape(4, 8, _VOCAB) for i in range(_N_CAT)
    ]
    return _tabular_embedding(idx_list, tab_list)
